# baseline (speedup 1.0000x reference)
import functools

import jax
import jax.numpy as jnp
from jax import lax
from jax.experimental import pallas as pl
from jax.experimental.pallas import tpu as pltpu
from jax.experimental.pallas import tpu_sc as plsc

N = 10000
D = 128
E = 320000
NC = 2
NS = 16
NW = NC * NS
CHUNK = 128
K = 80
EP = NW * K * CHUNK
NBUF = 4
N_PAD = 10240
RPT = N_PAD // NS
RB = RPT // CHUNK


def _mesh():
    return plsc.VectorSubcoreMesh(core_axis_name="c", subcore_axis_name="s")


def _sc_scatter(hp, src_r, dst_r, zeros_row):
    @functools.partial(
        pl.kernel,
        out_type=jax.ShapeDtypeStruct((NC, N_PAD, D), jnp.float32),
        mesh=_mesh(),
        scratch_types=[
            pltpu.VMEM((K, CHUNK), jnp.int32),
            pltpu.VMEM((K, CHUNK), jnp.int32),
            pltpu.VMEM((CHUNK, D), jnp.float32),
            pltpu.VMEM_SHARED((N_PAD, D), jnp.float32),
            pltpu.SemaphoreType.DMA,
        ],
    )
    def scat_k(h_hbm, src_hbm, dst_hbm, zeros_hbm, out_hbm,
               src_v, dst_v, rows_v, acc_sh, sem):
        c = lax.axis_index("c")
        s = lax.axis_index("s")
        wid = c * NS + s
        pltpu.sync_copy(src_hbm.at[wid], src_v)
        pltpu.sync_copy(dst_hbm.at[wid], dst_v)
        pltpu.sync_copy(zeros_hbm, rows_v)
        for j in range(RB):
            pltpu.sync_copy(rows_v,
                            acc_sh.at[pl.ds(s * RPT + j * CHUNK, CHUNK)])
        plsc.subcore_barrier()

        def body(k, carry):
            pltpu.async_copy(h_hbm.at[src_v.at[k]], rows_v, sem).wait()
            pltpu.sync_copy(rows_v, acc_sh.at[dst_v.at[k]], add=True)
            return carry

        lax.fori_loop(0, K, body, 0)
        plsc.subcore_barrier()
        for j in range(RB):
            pltpu.sync_copy(acc_sh.at[pl.ds(s * RPT + j * CHUNK, CHUNK)],
                            rows_v)
            pltpu.sync_copy(rows_v,
                            out_hbm.at[c, pl.ds(s * RPT + j * CHUNK, CHUNK)])

    return scat_k(hp, src_r, dst_r, zeros_row)


DW = 32


def _sc_deg(dst_r, ones_row, zeros_row):
    @functools.partial(
        pl.kernel,
        out_type=jax.ShapeDtypeStruct((NC, N_PAD, DW), jnp.float32),
        mesh=_mesh(),
        scratch_types=[
            pltpu.VMEM((K, CHUNK), jnp.int32),
            pltpu.VMEM((CHUNK, DW), jnp.float32),
            pltpu.VMEM((CHUNK, DW), jnp.float32),
            pltpu.VMEM_SHARED((N_PAD, DW), jnp.float32),
            [pltpu.SemaphoreType.DMA] * NBUF,
        ],
    )
    def deg_k(dst_hbm, ones_hbm, zeros_hbm, out_hbm,
              dst_v, ones_v, stage_v, acc_sh, sems):
        c = lax.axis_index("c")
        s = lax.axis_index("s")
        wid = c * NS + s
        pltpu.sync_copy(dst_hbm.at[wid], dst_v)
        pltpu.sync_copy(ones_hbm, ones_v)
        pltpu.sync_copy(zeros_hbm, stage_v)
        for j in range(RB):
            pltpu.sync_copy(stage_v,
                            acc_sh.at[pl.ds(s * RPT + j * CHUNK, CHUNK)])
        plsc.subcore_barrier()

        def body(q, carry):
            base = q * NBUF
            for j in range(NBUF):
                pltpu.async_copy(ones_v, acc_sh.at[dst_v.at[base + j]],
                                 sems[j], add=True)
            for j in range(NBUF):
                pltpu.make_async_copy(
                    ones_v, acc_sh.at[dst_v.at[0]], sems[j]).wait()
            return carry

        lax.fori_loop(0, K // NBUF, body, 0)
        plsc.subcore_barrier()
        for j in range(RB):
            pltpu.sync_copy(acc_sh.at[pl.ds(s * RPT + j * CHUNK, CHUNK)],
                            stage_v)
            pltpu.sync_copy(stage_v,
                            out_hbm.at[c, pl.ds(s * RPT + j * CHUNK, CHUNK)])

    return deg_k(dst_r, ones_row, zeros_row)


def _tc_pre_body(deg_ref, x_ref, w_ref, dis_ref, hp_ref):
    d = deg_ref[0, :N, 0:1] + deg_ref[1, :N, 0:1] + 1.0
    dis = lax.rsqrt(d)
    dis_ref[...] = dis
    hp_ref[...] = jnp.dot(x_ref[...], w_ref[...],
                          preferred_element_type=jnp.float32) * dis


def _tc_pre(deg_parts, x, W1):
    return pl.pallas_call(
        _tc_pre_body,
        out_shape=(
            jax.ShapeDtypeStruct((N, 1), jnp.float32),
            jax.ShapeDtypeStruct((N, D), jnp.float32),
        ),
    )(deg_parts, x, W1)


def _tc_mid_body(s_ref, hp_ref, dis_ref, b_ref, w_ref, out_ref):
    t = s_ref[0, :N, :] + s_ref[1, :N, :] + hp_ref[...]
    t = t * dis_ref[...] + b_ref[...]
    h = jnp.maximum(t, 0.0)
    out_ref[...] = jnp.dot(h, w_ref[...],
                           preferred_element_type=jnp.float32) * dis_ref[...]


def _tc_mid(s_parts, hp, dis, b, Wn):
    return pl.pallas_call(
        _tc_mid_body,
        out_shape=jax.ShapeDtypeStruct((N, D), jnp.float32),
    )(s_parts, hp, dis, b, Wn)


def _tc_fin_body(s_ref, hp_ref, dis_ref, b_ref, out_ref):
    t = s_ref[0, :N, :] + s_ref[1, :N, :] + hp_ref[...]
    out_ref[...] = t * dis_ref[...] + b_ref[...]


def _tc_fin(s_parts, hp, dis, b):
    return pl.pallas_call(
        _tc_fin_body,
        out_shape=jax.ShapeDtypeStruct((N, D), jnp.float32),
    )(s_parts, hp, dis, b)


def kernel(x, edge_index, W1, b1, W2, b2, W3, b3):
    src = edge_index[0].astype(jnp.int32)
    dst = edge_index[1].astype(jnp.int32)
    pad = EP - E
    src_p = jnp.concatenate([src, jnp.zeros((pad,), jnp.int32)])
    dst_p = jnp.concatenate([dst, jnp.full((pad,), N, jnp.int32)])
    src_r = src_p.reshape(NW, K, CHUNK)
    dst_r = dst_p.reshape(NW, K, CHUNK)

    zeros_row = jnp.zeros((CHUNK, D), jnp.float32)

    ones_tab = jnp.ones((N, D), jnp.float32)
    deg_parts = _sc_scatter(ones_tab, src_r, dst_r, zeros_row)
    dis, h1p = _tc_pre(deg_parts, x, W1)
    s1 = _sc_scatter(h1p, src_r, dst_r, zeros_row)
    h2p = _tc_mid(s1, h1p, dis, b1.reshape(1, D), W2)
    s2 = _sc_scatter(h2p, src_r, dst_r, zeros_row)
    h3p = _tc_mid(s2, h2p, dis, b2.reshape(1, D), W3)
    s3 = _sc_scatter(h3p, src_r, dst_r, zeros_row)
    z = _tc_fin(s3, h3p, dis, b3.reshape(1, D))
    return z


# speedup vs baseline: 1.5175x; 1.5175x over previous
import functools

import jax
import jax.numpy as jnp
from jax import lax
from jax.experimental import pallas as pl
from jax.experimental.pallas import tpu as pltpu
from jax.experimental.pallas import tpu_sc as plsc

N = 10000
D = 128
E = 320000
NC = 2
NS = 16
NW = NC * NS
CHUNK = 128
K = 79
EP = NW * K * CHUNK
NBUF = 4
N_PAD = 10240
RPT = N_PAD // NS
RB = RPT // CHUNK


def _mesh():
    return plsc.VectorSubcoreMesh(core_axis_name="c", subcore_axis_name="s")


def _sc_scatter(hp, src_r, dst_r, zeros_row):
    @functools.partial(
        pl.kernel,
        out_type=jax.ShapeDtypeStruct((NC, N_PAD, D), jnp.float32),
        mesh=_mesh(),
        scratch_types=[
            pltpu.VMEM((K, CHUNK), jnp.int32),
            pltpu.VMEM((K, CHUNK), jnp.int32),
            pltpu.VMEM((CHUNK, D), jnp.float32),
            pltpu.VMEM_SHARED((N_PAD, D), jnp.float32),
            pltpu.SemaphoreType.DMA,
        ],
    )
    def scat_k(h_hbm, src_hbm, dst_hbm, zeros_hbm, out_hbm,
               src_v, dst_v, rows_v, acc_sh, sem):
        c = lax.axis_index("c")
        s = lax.axis_index("s")
        wid = c * NS + s
        pltpu.sync_copy(src_hbm.at[wid], src_v)
        pltpu.sync_copy(dst_hbm.at[wid], dst_v)
        pltpu.sync_copy(zeros_hbm, rows_v)
        for j in range(RB):
            pltpu.sync_copy(rows_v,
                            acc_sh.at[pl.ds(s * RPT + j * CHUNK, CHUNK)])
        plsc.subcore_barrier()

        def body(k, carry):
            pltpu.async_copy(h_hbm.at[src_v.at[k]], rows_v, sem).wait()
            pltpu.sync_copy(rows_v, acc_sh.at[dst_v.at[k]], add=True)
            return carry

        lax.fori_loop(0, K, body, 0)
        plsc.subcore_barrier()
        for j in range(RB):
            pltpu.sync_copy(acc_sh.at[pl.ds(s * RPT + j * CHUNK, CHUNK)],
                            rows_v)
            pltpu.sync_copy(rows_v,
                            out_hbm.at[c, pl.ds(s * RPT + j * CHUNK, CHUNK)])

    return scat_k(hp, src_r, dst_r, zeros_row)


DW = 32


def _sc_deg(dst_r, ones_row, zeros_row):
    @functools.partial(
        pl.kernel,
        out_type=jax.ShapeDtypeStruct((NC, N_PAD, DW), jnp.float32),
        mesh=_mesh(),
        scratch_types=[
            pltpu.VMEM((K, CHUNK), jnp.int32),
            pltpu.VMEM((CHUNK, DW), jnp.float32),
            pltpu.VMEM((CHUNK, DW), jnp.float32),
            pltpu.VMEM_SHARED((N_PAD, DW), jnp.float32),
            [pltpu.SemaphoreType.DMA] * NBUF,
        ],
    )
    def deg_k(dst_hbm, ones_hbm, zeros_hbm, out_hbm,
              dst_v, ones_v, stage_v, acc_sh, sems):
        c = lax.axis_index("c")
        s = lax.axis_index("s")
        wid = c * NS + s
        pltpu.sync_copy(dst_hbm.at[wid], dst_v)
        pltpu.sync_copy(ones_hbm, ones_v)
        pltpu.sync_copy(zeros_hbm, stage_v)
        for j in range(RB):
            pltpu.sync_copy(stage_v,
                            acc_sh.at[pl.ds(s * RPT + j * CHUNK, CHUNK)])
        plsc.subcore_barrier()

        def body(q, carry):
            base = q * NBUF
            for j in range(NBUF):
                pltpu.async_copy(ones_v, acc_sh.at[dst_v.at[base + j]],
                                 sems[j], add=True)
            for j in range(NBUF):
                pltpu.make_async_copy(
                    ones_v, acc_sh.at[dst_v.at[0]], sems[j]).wait()
            return carry

        lax.fori_loop(0, K // NBUF, body, 0)
        plsc.subcore_barrier()
        for j in range(RB):
            pltpu.sync_copy(acc_sh.at[pl.ds(s * RPT + j * CHUNK, CHUNK)],
                            stage_v)
            pltpu.sync_copy(stage_v,
                            out_hbm.at[c, pl.ds(s * RPT + j * CHUNK, CHUNK)])

    return deg_k(dst_r, ones_row, zeros_row)


def _tc_pre_body(deg_ref, x_ref, w_ref, dis_ref, hp_ref):
    d = deg_ref[0, :N, 0:1] + deg_ref[1, :N, 0:1] + 1.0
    dis = lax.rsqrt(d)
    dis_ref[...] = dis
    hp_ref[...] = jnp.dot(x_ref[...], w_ref[...],
                          preferred_element_type=jnp.float32) * dis


def _tc_pre(deg_parts, x, W1):
    return pl.pallas_call(
        _tc_pre_body,
        out_shape=(
            jax.ShapeDtypeStruct((N, 1), jnp.float32),
            jax.ShapeDtypeStruct((N, D), jnp.float32),
        ),
    )(deg_parts, x, W1)


def _tc_mid_body(s_ref, hp_ref, dis_ref, b_ref, w_ref, out_ref):
    t = s_ref[0, :N, :] + s_ref[1, :N, :] + hp_ref[...]
    t = t * dis_ref[...] + b_ref[...]
    h = jnp.maximum(t, 0.0)
    out_ref[...] = jnp.dot(h, w_ref[...],
                           preferred_element_type=jnp.float32) * dis_ref[...]


def _tc_mid(s_parts, hp, dis, b, Wn):
    return pl.pallas_call(
        _tc_mid_body,
        out_shape=jax.ShapeDtypeStruct((N, D), jnp.float32),
    )(s_parts, hp, dis, b, Wn)


def _tc_fin_body(s_ref, hp_ref, dis_ref, b_ref, out_ref):
    t = s_ref[0, :N, :] + s_ref[1, :N, :] + hp_ref[...]
    out_ref[...] = t * dis_ref[...] + b_ref[...]


def _tc_fin(s_parts, hp, dis, b):
    return pl.pallas_call(
        _tc_fin_body,
        out_shape=jax.ShapeDtypeStruct((N, D), jnp.float32),
    )(s_parts, hp, dis, b)


def kernel(x, edge_index, W1, b1, W2, b2, W3, b3):
    src = edge_index[0].astype(jnp.int32)
    dst = edge_index[1].astype(jnp.int32)
    pad = EP - E
    src_p = jnp.concatenate([src, jnp.zeros((pad,), jnp.int32)])
    dst_p = jnp.concatenate([dst, jnp.full((pad,), N, jnp.int32)])
    src_r = src_p.reshape(NW, K, CHUNK)
    dst_r = dst_p.reshape(NW, K, CHUNK)

    zeros_row = jnp.zeros((CHUNK, D), jnp.float32)

    ones_tab = jnp.ones((N, D), jnp.float32)
    deg_parts = _sc_scatter(ones_tab, src_r, dst_r, zeros_row)
    dis, h1p = _tc_pre(deg_parts, x, W1)
    s1 = _sc_scatter(h1p, src_r, dst_r, zeros_row)
    h2p = _tc_mid(s1, h1p, dis, b1.reshape(1, D), W2)
    s2 = _sc_scatter(h2p, src_r, dst_r, zeros_row)
    h3p = _tc_mid(s2, h2p, dis, b2.reshape(1, D), W3)
    s3 = _sc_scatter(h3p, src_r, dst_r, zeros_row)
    z = _tc_fin(s3, h3p, dis, b3.reshape(1, D))
    return z


# speedup vs baseline: 3.1074x; 2.0477x over previous
import functools

import jax
import jax.numpy as jnp
from jax import lax
from jax.experimental import pallas as pl
from jax.experimental.pallas import tpu as pltpu
from jax.experimental.pallas import tpu_sc as plsc

N = 10000
D = 128
E = 320000
NC = 2
NS = 16
NW = NC * NS
CHUNK = 128
K = 80
EP = NW * K * CHUNK
NBUF = 4
N_PAD = 10240
RPT = N_PAD // NS
RB = RPT // CHUNK


def _mesh():
    return plsc.VectorSubcoreMesh(core_axis_name="c", subcore_axis_name="s")


def _sc_scatter(hp, src_r, dst_r, zeros_row):
    @functools.partial(
        pl.kernel,
        out_type=jax.ShapeDtypeStruct((NC, N_PAD, D), jnp.float32),
        mesh=_mesh(),
        scratch_types=[
            pltpu.VMEM((K, CHUNK), jnp.int32),
            pltpu.VMEM((K, CHUNK), jnp.int32),
            pltpu.VMEM((CHUNK, D), jnp.float32),
            pltpu.VMEM_SHARED((N_PAD, D), jnp.float32),
            pltpu.SemaphoreType.DMA,
        ],
    )
    def scat_k(h_hbm, src_hbm, dst_hbm, zeros_hbm, out_hbm,
               src_v, dst_v, rows_v, acc_sh, sem):
        c = lax.axis_index("c")
        s = lax.axis_index("s")
        wid = c * NS + s
        pltpu.sync_copy(src_hbm.at[wid], src_v)
        pltpu.sync_copy(dst_hbm.at[wid], dst_v)
        pltpu.sync_copy(zeros_hbm, rows_v)
        for j in range(RB):
            pltpu.sync_copy(rows_v,
                            acc_sh.at[pl.ds(s * RPT + j * CHUNK, CHUNK)])
        plsc.subcore_barrier()

        def body(k, carry):
            pltpu.async_copy(h_hbm.at[src_v.at[k]], rows_v, sem).wait()
            pltpu.sync_copy(rows_v, acc_sh.at[dst_v.at[k]], add=True)
            return carry

        lax.fori_loop(0, K, body, 0)
        plsc.subcore_barrier()
        for j in range(RB):
            pltpu.sync_copy(acc_sh.at[pl.ds(s * RPT + j * CHUNK, CHUNK)],
                            rows_v)
            pltpu.sync_copy(rows_v,
                            out_hbm.at[c, pl.ds(s * RPT + j * CHUNK, CHUNK)])

    return scat_k(hp, src_r, dst_r, zeros_row)


DW = 32


def _sc_deg(dst_r, ones_row, zeros_row):
    @functools.partial(
        pl.kernel,
        out_type=jax.ShapeDtypeStruct((NC, N_PAD, DW), jnp.float32),
        mesh=_mesh(),
        scratch_types=[
            pltpu.VMEM((K, CHUNK), jnp.int32),
            pltpu.VMEM((CHUNK, DW), jnp.float32),
            pltpu.VMEM((CHUNK, DW), jnp.float32),
            pltpu.VMEM_SHARED((N_PAD, DW), jnp.float32),
            [pltpu.SemaphoreType.DMA] * NBUF,
        ],
    )
    def deg_k(dst_hbm, ones_hbm, zeros_hbm, out_hbm,
              dst_v, ones_v, stage_v, acc_sh, sems):
        c = lax.axis_index("c")
        s = lax.axis_index("s")
        wid = c * NS + s
        pltpu.sync_copy(dst_hbm.at[wid], dst_v)
        pltpu.sync_copy(ones_hbm, ones_v)
        pltpu.sync_copy(zeros_hbm, stage_v)
        for j in range(RB):
            pltpu.sync_copy(stage_v,
                            acc_sh.at[pl.ds(s * RPT + j * CHUNK, CHUNK)])
        plsc.subcore_barrier()

        def body(q, carry):
            base = q * NBUF
            for j in range(NBUF):
                pltpu.async_copy(ones_v, acc_sh.at[dst_v.at[base + j]],
                                 sems[j], add=True)
            for j in range(NBUF):
                pltpu.make_async_copy(
                    ones_v, acc_sh.at[dst_v.at[0]], sems[j]).wait()
            return carry

        lax.fori_loop(0, K // NBUF, body, 0)
        plsc.subcore_barrier()
        for j in range(RB):
            pltpu.sync_copy(acc_sh.at[pl.ds(s * RPT + j * CHUNK, CHUNK)],
                            stage_v)
            pltpu.sync_copy(stage_v,
                            out_hbm.at[c, pl.ds(s * RPT + j * CHUNK, CHUNK)])

    return deg_k(dst_r, ones_row, zeros_row)


def _sc_deg(dst_r, ones_row, zeros_row):
    @functools.partial(
        pl.kernel,
        out_type=jax.ShapeDtypeStruct((NC, N_PAD, D), jnp.float32),
        mesh=_mesh(),
        scratch_types=[
            pltpu.VMEM((K, CHUNK), jnp.int32),
            pltpu.VMEM((CHUNK, D), jnp.float32),
            pltpu.VMEM_SHARED((N_PAD, D), jnp.float32),
        ],
    )
    def deg_k(dst_hbm, ones_hbm, zeros_hbm, out_hbm,
              dst_v, ones_v, acc_sh):
        c = lax.axis_index("c")
        s = lax.axis_index("s")
        wid = c * NS + s
        pltpu.sync_copy(dst_hbm.at[wid], dst_v)
        pltpu.sync_copy(ones_hbm, ones_v)
        for j in range(RB):
            pltpu.sync_copy(zeros_hbm,
                            acc_sh.at[pl.ds(s * RPT + j * CHUNK, CHUNK)])
        plsc.subcore_barrier()

        def body(k, carry):
            pltpu.sync_copy(ones_v, acc_sh.at[dst_v.at[k]], add=True)
            return carry

        lax.fori_loop(0, K, body, 0)
        plsc.subcore_barrier()
        for j in range(RB):
            pltpu.sync_copy(acc_sh.at[pl.ds(s * RPT + j * CHUNK, CHUNK)],
                            out_hbm.at[c, pl.ds(s * RPT + j * CHUNK, CHUNK)])

    return deg_k(dst_r, ones_row, zeros_row)

def _tc_pre_body(deg_ref, x_ref, w_ref, dis_ref, hp_ref):
    d = deg_ref[0, :N, 0:1] + deg_ref[1, :N, 0:1] + 1.0
    dis = lax.rsqrt(d)
    dis_ref[...] = dis
    hp_ref[...] = jnp.dot(x_ref[...], w_ref[...],
                          preferred_element_type=jnp.float32) * dis


def _tc_pre(deg_parts, x, W1):
    return pl.pallas_call(
        _tc_pre_body,
        out_shape=(
            jax.ShapeDtypeStruct((N, 1), jnp.float32),
            jax.ShapeDtypeStruct((N, D), jnp.float32),
        ),
    )(deg_parts, x, W1)


def _tc_mid_body(s_ref, hp_ref, dis_ref, b_ref, w_ref, out_ref):
    t = s_ref[0, :N, :] + s_ref[1, :N, :] + hp_ref[...]
    t = t * dis_ref[...] + b_ref[...]
    h = jnp.maximum(t, 0.0)
    out_ref[...] = jnp.dot(h, w_ref[...],
                           preferred_element_type=jnp.float32) * dis_ref[...]


def _tc_mid(s_parts, hp, dis, b, Wn):
    return pl.pallas_call(
        _tc_mid_body,
        out_shape=jax.ShapeDtypeStruct((N, D), jnp.float32),
    )(s_parts, hp, dis, b, Wn)


def _tc_fin_body(s_ref, hp_ref, dis_ref, b_ref, out_ref):
    t = s_ref[0, :N, :] + s_ref[1, :N, :] + hp_ref[...]
    out_ref[...] = t * dis_ref[...] + b_ref[...]


def _tc_fin(s_parts, hp, dis, b):
    return pl.pallas_call(
        _tc_fin_body,
        out_shape=jax.ShapeDtypeStruct((N, D), jnp.float32),
    )(s_parts, hp, dis, b)


def kernel(x, edge_index, W1, b1, W2, b2, W3, b3):
    src = edge_index[0].astype(jnp.int32)
    dst = edge_index[1].astype(jnp.int32)
    pad = EP - E
    ar = jnp.arange(pad, dtype=jnp.int32)
    src_p = jnp.concatenate([src, ar % N])
    dst_p = jnp.concatenate([dst, N + ar % (N_PAD - N)])
    src_r = src_p.reshape(NW, K, CHUNK)
    dst_r = dst_p.reshape(NW, K, CHUNK)

    zeros_row = jnp.zeros((CHUNK, D), jnp.float32)

    ones_row = jnp.ones((CHUNK, D), jnp.float32)
    deg_parts = _sc_deg(dst_r, ones_row, zeros_row)
    dis, h1p = _tc_pre(deg_parts, x, W1)
    s1 = _sc_scatter(h1p, src_r, dst_r, zeros_row)
    h2p = _tc_mid(s1, h1p, dis, b1.reshape(1, D), W2)
    s2 = _sc_scatter(h2p, src_r, dst_r, zeros_row)
    h3p = _tc_mid(s2, h2p, dis, b2.reshape(1, D), W3)
    s3 = _sc_scatter(h3p, src_r, dst_r, zeros_row)
    z = _tc_fin(s3, h3p, dis, b3.reshape(1, D))
    return z


# speedup vs baseline: 3.1219x; 1.0047x over previous
import functools

import jax
import jax.numpy as jnp
from jax import lax
from jax.experimental import pallas as pl
from jax.experimental.pallas import tpu as pltpu
from jax.experimental.pallas import tpu_sc as plsc

N = 10000
D = 128
E = 320000
NC = 2
NS = 16
NW = NC * NS
CHUNK = 128
K = 80
EP = NW * K * CHUNK
NBUF = 4
N_PAD = 10240
RPT = N_PAD // NS
RB = RPT // CHUNK


def _mesh():
    return plsc.VectorSubcoreMesh(core_axis_name="c", subcore_axis_name="s")


def _sc_scatter(hp, src_r, dst_r, zeros_row):
    @functools.partial(
        pl.kernel,
        out_type=jax.ShapeDtypeStruct((NC, N_PAD, D), jnp.float32),
        mesh=_mesh(),
        scratch_types=[
            pltpu.VMEM((K, CHUNK), jnp.int32),
            pltpu.VMEM((K, CHUNK), jnp.int32),
            pltpu.VMEM((CHUNK, D), jnp.float32),
            pltpu.VMEM_SHARED((N_PAD, D), jnp.float32),
            pltpu.SemaphoreType.DMA,
        ],
    )
    def scat_k(h_hbm, src_hbm, dst_hbm, zeros_hbm, out_hbm,
               src_v, dst_v, rows_v, acc_sh, sem):
        c = lax.axis_index("c")
        s = lax.axis_index("s")
        wid = c * NS + s
        pltpu.sync_copy(src_hbm.at[wid], src_v)
        pltpu.sync_copy(dst_hbm.at[wid], dst_v)
        pltpu.sync_copy(zeros_hbm, rows_v)
        for j in range(RB):
            pltpu.sync_copy(rows_v,
                            acc_sh.at[pl.ds(s * RPT + j * CHUNK, CHUNK)])
        plsc.subcore_barrier()

        def body(k, carry):
            pltpu.async_copy(h_hbm.at[src_v.at[k]], rows_v, sem).wait()
            pltpu.sync_copy(rows_v, acc_sh.at[dst_v.at[k]], add=True)
            return carry

        lax.fori_loop(0, K, body, 0)
        plsc.subcore_barrier()
        for j in range(RB):
            pltpu.sync_copy(acc_sh.at[pl.ds(s * RPT + j * CHUNK, CHUNK)],
                            out_hbm.at[c, pl.ds(s * RPT + j * CHUNK, CHUNK)])

    return scat_k(hp, src_r, dst_r, zeros_row)


DW = 32


def _sc_deg(dst_r, ones_row, zeros_row):
    @functools.partial(
        pl.kernel,
        out_type=jax.ShapeDtypeStruct((NC, N_PAD, DW), jnp.float32),
        mesh=_mesh(),
        scratch_types=[
            pltpu.VMEM((K, CHUNK), jnp.int32),
            pltpu.VMEM((CHUNK, DW), jnp.float32),
            pltpu.VMEM((CHUNK, DW), jnp.float32),
            pltpu.VMEM_SHARED((N_PAD, DW), jnp.float32),
            [pltpu.SemaphoreType.DMA] * NBUF,
        ],
    )
    def deg_k(dst_hbm, ones_hbm, zeros_hbm, out_hbm,
              dst_v, ones_v, stage_v, acc_sh, sems):
        c = lax.axis_index("c")
        s = lax.axis_index("s")
        wid = c * NS + s
        pltpu.sync_copy(dst_hbm.at[wid], dst_v)
        pltpu.sync_copy(ones_hbm, ones_v)
        pltpu.sync_copy(zeros_hbm, stage_v)
        for j in range(RB):
            pltpu.sync_copy(stage_v,
                            acc_sh.at[pl.ds(s * RPT + j * CHUNK, CHUNK)])
        plsc.subcore_barrier()

        def body(q, carry):
            base = q * NBUF
            for j in range(NBUF):
                pltpu.async_copy(ones_v, acc_sh.at[dst_v.at[base + j]],
                                 sems[j], add=True)
            for j in range(NBUF):
                pltpu.make_async_copy(
                    ones_v, acc_sh.at[dst_v.at[0]], sems[j]).wait()
            return carry

        lax.fori_loop(0, K // NBUF, body, 0)
        plsc.subcore_barrier()
        for j in range(RB):
            pltpu.sync_copy(acc_sh.at[pl.ds(s * RPT + j * CHUNK, CHUNK)],
                            stage_v)
            pltpu.sync_copy(stage_v,
                            out_hbm.at[c, pl.ds(s * RPT + j * CHUNK, CHUNK)])

    return deg_k(dst_r, ones_row, zeros_row)


def _sc_deg(dst_r, ones_row, zeros_row):
    @functools.partial(
        pl.kernel,
        out_type=jax.ShapeDtypeStruct((NC, N_PAD, D), jnp.float32),
        mesh=_mesh(),
        scratch_types=[
            pltpu.VMEM((K, CHUNK), jnp.int32),
            pltpu.VMEM((CHUNK, D), jnp.float32),
            pltpu.VMEM_SHARED((N_PAD, D), jnp.float32),
        ],
    )
    def deg_k(dst_hbm, ones_hbm, zeros_hbm, out_hbm,
              dst_v, ones_v, acc_sh):
        c = lax.axis_index("c")
        s = lax.axis_index("s")
        wid = c * NS + s
        pltpu.sync_copy(dst_hbm.at[wid], dst_v)
        pltpu.sync_copy(ones_hbm, ones_v)
        for j in range(RB):
            pltpu.sync_copy(zeros_hbm,
                            acc_sh.at[pl.ds(s * RPT + j * CHUNK, CHUNK)])
        plsc.subcore_barrier()

        def body(k, carry):
            pltpu.sync_copy(ones_v, acc_sh.at[dst_v.at[k]], add=True)
            return carry

        lax.fori_loop(0, K, body, 0)
        plsc.subcore_barrier()
        for j in range(RB):
            pltpu.sync_copy(acc_sh.at[pl.ds(s * RPT + j * CHUNK, CHUNK)],
                            out_hbm.at[c, pl.ds(s * RPT + j * CHUNK, CHUNK)])

    return deg_k(dst_r, ones_row, zeros_row)

def _tc_pre_body(deg_ref, x_ref, w_ref, dis_ref, hp_ref):
    d = deg_ref[0, :N, 0:1] + deg_ref[1, :N, 0:1] + 1.0
    dis = lax.rsqrt(d)
    dis_ref[...] = dis
    hp_ref[...] = jnp.dot(x_ref[...], w_ref[...],
                          preferred_element_type=jnp.float32) * dis


def _tc_pre(deg_parts, x, W1):
    return pl.pallas_call(
        _tc_pre_body,
        out_shape=(
            jax.ShapeDtypeStruct((N, 1), jnp.float32),
            jax.ShapeDtypeStruct((N, D), jnp.float32),
        ),
    )(deg_parts, x, W1)


def _tc_mid_body(s_ref, hp_ref, dis_ref, b_ref, w_ref, out_ref):
    t = s_ref[0, :N, :] + s_ref[1, :N, :] + hp_ref[...]
    t = t * dis_ref[...] + b_ref[...]
    h = jnp.maximum(t, 0.0)
    out_ref[...] = jnp.dot(h, w_ref[...],
                           preferred_element_type=jnp.float32) * dis_ref[...]


def _tc_mid(s_parts, hp, dis, b, Wn):
    return pl.pallas_call(
        _tc_mid_body,
        out_shape=jax.ShapeDtypeStruct((N, D), jnp.float32),
    )(s_parts, hp, dis, b, Wn)


def _tc_fin_body(s_ref, hp_ref, dis_ref, b_ref, out_ref):
    t = s_ref[0, :N, :] + s_ref[1, :N, :] + hp_ref[...]
    out_ref[...] = t * dis_ref[...] + b_ref[...]


def _tc_fin(s_parts, hp, dis, b):
    return pl.pallas_call(
        _tc_fin_body,
        out_shape=jax.ShapeDtypeStruct((N, D), jnp.float32),
    )(s_parts, hp, dis, b)


def kernel(x, edge_index, W1, b1, W2, b2, W3, b3):
    src = edge_index[0].astype(jnp.int32)
    dst = edge_index[1].astype(jnp.int32)
    pad = EP - E
    ar = jnp.arange(pad, dtype=jnp.int32)
    src_p = jnp.concatenate([src, ar % N])
    dst_p = jnp.concatenate([dst, N + ar % (N_PAD - N)])
    src_r = src_p.reshape(NW, K, CHUNK)
    dst_r = dst_p.reshape(NW, K, CHUNK)

    zeros_row = jnp.zeros((CHUNK, D), jnp.float32)

    ones_row = jnp.ones((CHUNK, D), jnp.float32)
    deg_parts = _sc_deg(dst_r, ones_row, zeros_row)
    dis, h1p = _tc_pre(deg_parts, x, W1)
    s1 = _sc_scatter(h1p, src_r, dst_r, zeros_row)
    h2p = _tc_mid(s1, h1p, dis, b1.reshape(1, D), W2)
    s2 = _sc_scatter(h2p, src_r, dst_r, zeros_row)
    h3p = _tc_mid(s2, h2p, dis, b2.reshape(1, D), W3)
    s3 = _sc_scatter(h3p, src_r, dst_r, zeros_row)
    z = _tc_fin(s3, h3p, dis, b3.reshape(1, D))
    return z


# speedup vs baseline: 3.1574x; 1.0114x over previous
import functools

import jax
import jax.numpy as jnp
from jax import lax
from jax.experimental import pallas as pl
from jax.experimental.pallas import tpu as pltpu
from jax.experimental.pallas import tpu_sc as plsc

N = 10000
D = 128
E = 320000
NC = 2
NS = 16
NW = NC * NS
CHUNK = 128
K = 79
EP = NW * K * CHUNK
NBUF = 4
N_PAD = 10240
RPT = N_PAD // NS
RB = RPT // CHUNK


def _mesh():
    return plsc.VectorSubcoreMesh(core_axis_name="c", subcore_axis_name="s")


def _sc_scatter(hp, src_r, dst_r, zeros_row):
    @functools.partial(
        pl.kernel,
        out_type=jax.ShapeDtypeStruct((NC, N_PAD, D), jnp.float32),
        mesh=_mesh(),
        scratch_types=[
            pltpu.VMEM((K, CHUNK), jnp.int32),
            pltpu.VMEM((K, CHUNK), jnp.int32),
            pltpu.VMEM((CHUNK, D), jnp.float32),
            pltpu.VMEM_SHARED((N_PAD, D), jnp.float32),
            pltpu.SemaphoreType.DMA,
        ],
    )
    def scat_k(h_hbm, src_hbm, dst_hbm, zeros_hbm, out_hbm,
               src_v, dst_v, rows_v, acc_sh, sem):
        c = lax.axis_index("c")
        s = lax.axis_index("s")
        wid = c * NS + s
        pltpu.sync_copy(src_hbm.at[wid], src_v)
        pltpu.sync_copy(dst_hbm.at[wid], dst_v)
        pltpu.sync_copy(zeros_hbm, rows_v)
        for j in range(RB):
            pltpu.sync_copy(rows_v,
                            acc_sh.at[pl.ds(s * RPT + j * CHUNK, CHUNK)])
        plsc.subcore_barrier()

        def body(k, carry):
            pltpu.async_copy(h_hbm.at[src_v.at[k]], rows_v, sem).wait()
            pltpu.sync_copy(rows_v, acc_sh.at[dst_v.at[k]], add=True)
            return carry

        lax.fori_loop(0, K, body, 0)
        plsc.subcore_barrier()
        for j in range(RB):
            pltpu.sync_copy(acc_sh.at[pl.ds(s * RPT + j * CHUNK, CHUNK)],
                            out_hbm.at[c, pl.ds(s * RPT + j * CHUNK, CHUNK)])

    return scat_k(hp, src_r, dst_r, zeros_row)


DW = 32


def _sc_deg(dst_r, ones_row, zeros_row):
    @functools.partial(
        pl.kernel,
        out_type=jax.ShapeDtypeStruct((NC, N_PAD, DW), jnp.float32),
        mesh=_mesh(),
        scratch_types=[
            pltpu.VMEM((K, CHUNK), jnp.int32),
            pltpu.VMEM((CHUNK, DW), jnp.float32),
            pltpu.VMEM((CHUNK, DW), jnp.float32),
            pltpu.VMEM_SHARED((N_PAD, DW), jnp.float32),
            [pltpu.SemaphoreType.DMA] * NBUF,
        ],
    )
    def deg_k(dst_hbm, ones_hbm, zeros_hbm, out_hbm,
              dst_v, ones_v, stage_v, acc_sh, sems):
        c = lax.axis_index("c")
        s = lax.axis_index("s")
        wid = c * NS + s
        pltpu.sync_copy(dst_hbm.at[wid], dst_v)
        pltpu.sync_copy(ones_hbm, ones_v)
        pltpu.sync_copy(zeros_hbm, stage_v)
        for j in range(RB):
            pltpu.sync_copy(stage_v,
                            acc_sh.at[pl.ds(s * RPT + j * CHUNK, CHUNK)])
        plsc.subcore_barrier()

        def body(q, carry):
            base = q * NBUF
            for j in range(NBUF):
                pltpu.async_copy(ones_v, acc_sh.at[dst_v.at[base + j]],
                                 sems[j], add=True)
            for j in range(NBUF):
                pltpu.make_async_copy(
                    ones_v, acc_sh.at[dst_v.at[0]], sems[j]).wait()
            return carry

        lax.fori_loop(0, K // NBUF, body, 0)
        plsc.subcore_barrier()
        for j in range(RB):
            pltpu.sync_copy(acc_sh.at[pl.ds(s * RPT + j * CHUNK, CHUNK)],
                            stage_v)
            pltpu.sync_copy(stage_v,
                            out_hbm.at[c, pl.ds(s * RPT + j * CHUNK, CHUNK)])

    return deg_k(dst_r, ones_row, zeros_row)


def _sc_deg(dst_r, ones_row, zeros_row):
    @functools.partial(
        pl.kernel,
        out_type=jax.ShapeDtypeStruct((NC, N_PAD, D), jnp.float32),
        mesh=_mesh(),
        scratch_types=[
            pltpu.VMEM((K, CHUNK), jnp.int32),
            pltpu.VMEM((CHUNK, D), jnp.float32),
            pltpu.VMEM_SHARED((N_PAD, D), jnp.float32),
        ],
    )
    def deg_k(dst_hbm, ones_hbm, zeros_hbm, out_hbm,
              dst_v, ones_v, acc_sh):
        c = lax.axis_index("c")
        s = lax.axis_index("s")
        wid = c * NS + s
        pltpu.sync_copy(dst_hbm.at[wid], dst_v)
        pltpu.sync_copy(ones_hbm, ones_v)
        for j in range(RB):
            pltpu.sync_copy(zeros_hbm,
                            acc_sh.at[pl.ds(s * RPT + j * CHUNK, CHUNK)])
        plsc.subcore_barrier()

        def body(k, carry):
            pltpu.sync_copy(ones_v, acc_sh.at[dst_v.at[k]], add=True)
            return carry

        lax.fori_loop(0, K, body, 0)
        plsc.subcore_barrier()
        for j in range(RB):
            pltpu.sync_copy(acc_sh.at[pl.ds(s * RPT + j * CHUNK, CHUNK)],
                            out_hbm.at[c, pl.ds(s * RPT + j * CHUNK, CHUNK)])

    return deg_k(dst_r, ones_row, zeros_row)

def _tc_pre_body(deg_ref, x_ref, w_ref, dis_ref, hp_ref):
    d = deg_ref[0, :N, 0:1] + deg_ref[1, :N, 0:1] + 1.0
    dis = lax.rsqrt(d)
    dis_ref[...] = dis
    hp_ref[...] = jnp.dot(x_ref[...], w_ref[...],
                          preferred_element_type=jnp.float32) * dis


def _tc_pre(deg_parts, x, W1):
    return pl.pallas_call(
        _tc_pre_body,
        out_shape=(
            jax.ShapeDtypeStruct((N, 1), jnp.float32),
            jax.ShapeDtypeStruct((N, D), jnp.float32),
        ),
    )(deg_parts, x, W1)


def _tc_mid_body(s_ref, hp_ref, dis_ref, b_ref, w_ref, out_ref):
    t = s_ref[0, :N, :] + s_ref[1, :N, :] + hp_ref[...]
    t = t * dis_ref[...] + b_ref[...]
    h = jnp.maximum(t, 0.0)
    out_ref[...] = jnp.dot(h, w_ref[...],
                           preferred_element_type=jnp.float32) * dis_ref[...]


def _tc_mid(s_parts, hp, dis, b, Wn):
    return pl.pallas_call(
        _tc_mid_body,
        out_shape=jax.ShapeDtypeStruct((N, D), jnp.float32),
    )(s_parts, hp, dis, b, Wn)


def _tc_fin_body(s_ref, hp_ref, dis_ref, b_ref, out_ref):
    t = s_ref[0, :N, :] + s_ref[1, :N, :] + hp_ref[...]
    out_ref[...] = t * dis_ref[...] + b_ref[...]


def _tc_fin(s_parts, hp, dis, b):
    return pl.pallas_call(
        _tc_fin_body,
        out_shape=jax.ShapeDtypeStruct((N, D), jnp.float32),
    )(s_parts, hp, dis, b)


def kernel(x, edge_index, W1, b1, W2, b2, W3, b3):
    src = edge_index[0].astype(jnp.int32)
    dst = edge_index[1].astype(jnp.int32)
    pad = EP - E
    ar = jnp.arange(pad, dtype=jnp.int32)
    src_p = jnp.concatenate([src, ar % N])
    dst_p = jnp.concatenate([dst, N + ar % (N_PAD - N)])
    src_r = src_p.reshape(NW, K, CHUNK)
    dst_r = dst_p.reshape(NW, K, CHUNK)

    zeros_row = jnp.zeros((CHUNK, D), jnp.float32)

    ones_row = jnp.ones((CHUNK, D), jnp.float32)
    deg_parts = _sc_deg(dst_r, ones_row, zeros_row)
    dis, h1p = _tc_pre(deg_parts, x, W1)
    s1 = _sc_scatter(h1p, src_r, dst_r, zeros_row)
    h2p = _tc_mid(s1, h1p, dis, b1.reshape(1, D), W2)
    s2 = _sc_scatter(h2p, src_r, dst_r, zeros_row)
    h3p = _tc_mid(s2, h2p, dis, b2.reshape(1, D), W3)
    s3 = _sc_scatter(h3p, src_r, dst_r, zeros_row)
    z = _tc_fin(s3, h3p, dis, b3.reshape(1, D))
    return z
